# baseline (speedup 1.0000x reference)
import functools

import jax
import jax.numpy as jnp
from jax import lax
from jax.experimental import pallas as pl
from jax.experimental.pallas import tpu as pltpu
from jax.experimental.pallas import tpu_sc as plsc

VOCAB = 1000000
EMBED_DIM = 32
BATCH = 16384
SEQ = 20

NUM_CORES = 2
NUM_SUBCORES = 16
NUM_WORKERS = NUM_CORES * NUM_SUBCORES

B_TOTAL = BATCH * SEQ
B_PER_W = B_TOTAL // NUM_WORKERS
R_PER_W = BATCH // NUM_WORKERS
RB = 64
CHUNK = RB * SEQ
N_CHUNKS = B_PER_W // CHUNK


def _gather_body(idx_hbm, table_hbm, out_hbm, idx_v, rows_a, rows_b,
                 gsem_a, gsem_b, wsem):
    wid = lax.axis_index("s") * NUM_CORES + lax.axis_index("c")
    base = wid * B_PER_W
    brow0 = wid * R_PER_W
    pltpu.sync_copy(idx_hbm.at[pl.ds(base, B_PER_W)], idx_v)

    bufs = (rows_a, rows_b)
    gsems = (gsem_a, gsem_b)

    def start_writes(buf, c):
        b0 = brow0 + c * RB
        for k in range(RB):
            pltpu.async_copy(buf.at[pl.ds(k * SEQ, SEQ)], out_hbm.at[b0 + k],
                             wsem)

    def drain_writes():
        pltpu.make_async_copy(table_hbm.at[pl.ds(0, CHUNK)], rows_a,
                              wsem).wait()

    pltpu.async_copy(table_hbm.at[idx_v.at[pl.ds(0, CHUNK)]], rows_a, gsem_a)
    for c in range(N_CHUNKS):
        cur = bufs[c % 2]
        if c + 1 < N_CHUNKS:
            if c >= 1:
                drain_writes()
            pltpu.async_copy(
                table_hbm.at[idx_v.at[pl.ds((c + 1) * CHUNK, CHUNK)]],
                bufs[(c + 1) % 2], gsems[(c + 1) % 2])
        pltpu.make_async_copy(
            table_hbm.at[idx_v.at[pl.ds(c * CHUNK, CHUNK)]], cur,
            gsems[c % 2]).wait()
        start_writes(cur, c)
    drain_writes()
    drain_writes()


FMT_COLS = 16384
FMT_GRID = -(-VOCAB // FMT_COLS)


IMG_ROWS = 2048
IMG_BLK = 512
IMG_GRID = BATCH // IMG_BLK


def _fmt_body(tab_ref, img_ref, out_ref, img_out_ref):
    i = pl.program_id(0)

    @pl.when(i < FMT_GRID)
    def _table():
        x = tab_ref[...]
        y = jnp.transpose(x, (1, 0))
        y3 = y.reshape(FMT_COLS // 4, 4, EMBED_DIM)
        parts = [y3[:, q, :] for q in range(4)]
        out_ref[...] = jnp.concatenate(parts, axis=1)

    @pl.when(i >= FMT_GRID)
    def _image():
        img_out_ref[...] = img_ref[...]


@jax.jit
def _tc_format(table_t, image):
    ntab = FMT_GRID - 1
    return pl.pallas_call(
        _fmt_body,
        grid=(FMT_GRID + IMG_GRID,),
        in_specs=[
            pl.BlockSpec((EMBED_DIM, FMT_COLS),
                         lambda i: (0, jnp.minimum(i, ntab))),
            pl.BlockSpec((IMG_BLK, IMG_ROWS),
                         lambda i: (jnp.maximum(i - FMT_GRID, 0), 0)),
        ],
        out_specs=[
            pl.BlockSpec((FMT_COLS // 4, 128),
                         lambda i: (jnp.minimum(i, ntab), 0)),
            pl.BlockSpec((IMG_BLK, IMG_ROWS),
                         lambda i: (jnp.maximum(i - FMT_GRID, 0), 0)),
        ],
        out_shape=[
            jax.ShapeDtypeStruct((VOCAB // 4, 128), jnp.float32),
            jax.ShapeDtypeStruct((BATCH, IMG_ROWS), jnp.float32),
        ],
    )(table_t, image)


@jax.jit
def _sc_gather(idx_flat, table):
    mesh = plsc.VectorSubcoreMesh(core_axis_name="c", subcore_axis_name="s")
    return pl.kernel(
        _gather_body,
        out_type=jax.ShapeDtypeStruct((BATCH, SEQ, EMBED_DIM), jnp.float32),
        mesh=mesh,
        scratch_types=[
            pltpu.VMEM((B_PER_W,), jnp.int32),
            pltpu.VMEM((CHUNK, EMBED_DIM), jnp.float32),
            pltpu.VMEM((CHUNK, EMBED_DIM), jnp.float32),
            pltpu.SemaphoreType.DMA,
            pltpu.SemaphoreType.DMA,
            pltpu.SemaphoreType.DMA,
        ],
        compiler_params=pltpu.CompilerParams(use_tc_tiling_on_sc=False),
    )(idx_flat, table)


def kernel(image, question, table):
    idx_flat = question.reshape(-1).astype(jnp.int32)
    tbl_lin, image_out = _tc_format(table.swapaxes(0, 1), image)
    emb = _sc_gather(idx_flat, tbl_lin.reshape(VOCAB, EMBED_DIM))
    return (image_out, emb)


# speedup vs baseline: 1.0084x; 1.0084x over previous
import functools

import jax
import jax.numpy as jnp
from jax import lax
from jax.experimental import pallas as pl
from jax.experimental.pallas import tpu as pltpu
from jax.experimental.pallas import tpu_sc as plsc

VOCAB = 1000000
EMBED_DIM = 32
BATCH = 16384
SEQ = 20

NUM_CORES = 2
NUM_SUBCORES = 16
NUM_WORKERS = NUM_CORES * NUM_SUBCORES

B_TOTAL = BATCH * SEQ
B_PER_W = B_TOTAL // NUM_WORKERS
R_PER_W = BATCH // NUM_WORKERS
RB = 64
CHUNK = RB * SEQ
N_CHUNKS = B_PER_W // CHUNK


def _gather_body(idx_hbm, table_hbm, out_hbm, idx_v, rows_a, rows_b,
                 gsem_a, gsem_b, wsem):
    wid = lax.axis_index("s") * NUM_CORES + lax.axis_index("c")
    base = wid * B_PER_W
    brow0 = wid * R_PER_W
    pltpu.sync_copy(idx_hbm.at[pl.ds(base, B_PER_W)], idx_v)

    bufs = (rows_a, rows_b)
    gsems = (gsem_a, gsem_b)

    def start_writes(buf, c):
        b0 = brow0 + c * RB
        for k in range(RB):
            pltpu.async_copy(buf.at[pl.ds(k * SEQ, SEQ)], out_hbm.at[b0 + k],
                             wsem)

    def drain_writes():
        pltpu.make_async_copy(table_hbm.at[pl.ds(0, CHUNK)], rows_a,
                              wsem).wait()

    pltpu.async_copy(table_hbm.at[idx_v.at[pl.ds(0, CHUNK)]], rows_a, gsem_a)
    for c in range(N_CHUNKS):
        cur = bufs[c % 2]
        if c + 1 < N_CHUNKS:
            if c >= 1:
                drain_writes()
            pltpu.async_copy(
                table_hbm.at[idx_v.at[pl.ds((c + 1) * CHUNK, CHUNK)]],
                bufs[(c + 1) % 2], gsems[(c + 1) % 2])
        pltpu.make_async_copy(
            table_hbm.at[idx_v.at[pl.ds(c * CHUNK, CHUNK)]], cur,
            gsems[c % 2]).wait()
        start_writes(cur, c)
    drain_writes()
    drain_writes()


FMT_COLS = 16384
FMT_GRID = -(-VOCAB // FMT_COLS)


IMG_ROWS = 2048
IMG_BLK = 512
IMG_GRID = BATCH // IMG_BLK


def _fmt_body(tab_ref, out_ref):
    x = tab_ref[...]
    y = jnp.transpose(x, (1, 0))
    y3 = y.reshape(FMT_COLS // 4, 4, EMBED_DIM)
    parts = [y3[:, q, :] for q in range(4)]
    out_ref[...] = jnp.concatenate(parts, axis=1)


@jax.jit
def _tc_format(table_t):
    return pl.pallas_call(
        _fmt_body,
        grid=(FMT_GRID,),
        in_specs=[
            pl.BlockSpec((EMBED_DIM, FMT_COLS), lambda i: (0, i)),
        ],
        out_specs=pl.BlockSpec((FMT_COLS // 4, 128), lambda i: (i, 0)),
        out_shape=jax.ShapeDtypeStruct((VOCAB // 4, 128), jnp.float32),
    )(table_t)


@jax.jit
def _sc_gather(idx_flat, table):
    mesh = plsc.VectorSubcoreMesh(core_axis_name="c", subcore_axis_name="s")
    return pl.kernel(
        _gather_body,
        out_type=jax.ShapeDtypeStruct((BATCH, SEQ, EMBED_DIM), jnp.float32),
        mesh=mesh,
        scratch_types=[
            pltpu.VMEM((B_PER_W,), jnp.int32),
            pltpu.VMEM((CHUNK, EMBED_DIM), jnp.float32),
            pltpu.VMEM((CHUNK, EMBED_DIM), jnp.float32),
            pltpu.SemaphoreType.DMA,
            pltpu.SemaphoreType.DMA,
            pltpu.SemaphoreType.DMA,
        ],
        compiler_params=pltpu.CompilerParams(use_tc_tiling_on_sc=False),
    )(idx_flat, table)


def kernel(image, question, table):
    idx_flat = question.reshape(-1).astype(jnp.int32)
    tbl_lin = _tc_format(table.swapaxes(0, 1))
    emb = _sc_gather(idx_flat, tbl_lin.reshape(VOCAB, EMBED_DIM))
    return (image, emb)


# speedup vs baseline: 1.0118x; 1.0034x over previous
import functools

import jax
import jax.numpy as jnp
from jax import lax
from jax.experimental import pallas as pl
from jax.experimental.pallas import tpu as pltpu
from jax.experimental.pallas import tpu_sc as plsc

VOCAB = 1000000
EMBED_DIM = 32
BATCH = 16384
SEQ = 20

NUM_CORES = 2
NUM_SUBCORES = 16
NUM_WORKERS = NUM_CORES * NUM_SUBCORES

B_TOTAL = BATCH * SEQ
B_PER_W = B_TOTAL // NUM_WORKERS
R_PER_W = BATCH // NUM_WORKERS
RB = 64
CHUNK = RB * SEQ
N_CHUNKS = B_PER_W // CHUNK


def _gather_body(idx_hbm, table_hbm, out_hbm, idx_v, rows_a, rows_b,
                 gsem_a, gsem_b, wsem):
    wid = lax.axis_index("s") * NUM_CORES + lax.axis_index("c")
    base = wid * B_PER_W
    brow0 = wid * R_PER_W
    pltpu.sync_copy(idx_hbm.at[pl.ds(base, B_PER_W)], idx_v)

    bufs = (rows_a, rows_b)
    gsems = (gsem_a, gsem_b)

    def start_writes(buf, c):
        b0 = brow0 + c * RB
        for k in range(RB):
            pltpu.async_copy(buf.at[pl.ds(k * SEQ, SEQ)], out_hbm.at[b0 + k],
                             wsem)

    def drain_writes():
        pltpu.make_async_copy(table_hbm.at[pl.ds(0, CHUNK)], rows_a,
                              wsem).wait()

    pltpu.async_copy(table_hbm.at[idx_v.at[pl.ds(0, CHUNK)]], rows_a, gsem_a)
    for c in range(N_CHUNKS):
        cur = bufs[c % 2]
        if c + 1 < N_CHUNKS:
            if c >= 1:
                drain_writes()
            pltpu.async_copy(
                table_hbm.at[idx_v.at[pl.ds((c + 1) * CHUNK, CHUNK)]],
                bufs[(c + 1) % 2], gsems[(c + 1) % 2])
        pltpu.make_async_copy(
            table_hbm.at[idx_v.at[pl.ds(c * CHUNK, CHUNK)]], cur,
            gsems[c % 2]).wait()
        start_writes(cur, c)
    drain_writes()
    drain_writes()


FMT_COLS = 32768
FMT_GRID = -(-VOCAB // FMT_COLS)


IMG_ROWS = 2048
IMG_BLK = 512
IMG_GRID = BATCH // IMG_BLK


def _fmt_body(tab_ref, out_ref):
    x = tab_ref[...]
    y = jnp.transpose(x, (1, 0))
    y3 = y.reshape(FMT_COLS // 4, 4, EMBED_DIM)
    parts = [y3[:, q, :] for q in range(4)]
    out_ref[...] = jnp.concatenate(parts, axis=1)


@jax.jit
def _tc_format(table_t):
    return pl.pallas_call(
        _fmt_body,
        grid=(FMT_GRID,),
        in_specs=[
            pl.BlockSpec((EMBED_DIM, FMT_COLS), lambda i: (0, i)),
        ],
        out_specs=pl.BlockSpec((FMT_COLS // 4, 128), lambda i: (i, 0)),
        out_shape=jax.ShapeDtypeStruct((VOCAB // 4, 128), jnp.float32),
    )(table_t)


@jax.jit
def _sc_gather(idx_flat, table):
    mesh = plsc.VectorSubcoreMesh(core_axis_name="c", subcore_axis_name="s")
    return pl.kernel(
        _gather_body,
        out_type=jax.ShapeDtypeStruct((BATCH, SEQ, EMBED_DIM), jnp.float32),
        mesh=mesh,
        scratch_types=[
            pltpu.VMEM((B_PER_W,), jnp.int32),
            pltpu.VMEM((CHUNK, EMBED_DIM), jnp.float32),
            pltpu.VMEM((CHUNK, EMBED_DIM), jnp.float32),
            pltpu.SemaphoreType.DMA,
            pltpu.SemaphoreType.DMA,
            pltpu.SemaphoreType.DMA,
        ],
        compiler_params=pltpu.CompilerParams(use_tc_tiling_on_sc=False),
    )(idx_flat, table)


def kernel(image, question, table):
    idx_flat = question.reshape(-1).astype(jnp.int32)
    tbl_lin = _tc_format(table.swapaxes(0, 1))
    emb = _sc_gather(idx_flat, tbl_lin.reshape(VOCAB, EMBED_DIM))
    return (image, emb)
